# baseline (speedup 1.0000x reference)
import functools

import jax
import jax.numpy as jnp
from jax import lax
from jax.experimental import pallas as pl
from jax.experimental.pallas import tpu as pltpu
from jax.experimental.pallas import tpu_sc as plsc

D_MODEL = 64
NUM_CORES = 2
NUM_SUBCORES = 16
NUM_WORKERS = NUM_CORES * NUM_SUBCORES

B = 16384
T = 200
BHI = B // 128
THI = T // 8
PAIRS = T * BHI // 2
PAIRS_PER_W = PAIRS // NUM_WORKERS


def _build_lookup():
  mesh = plsc.VectorSubcoreMesh(core_axis_name="c", subcore_axis_name="s")

  @functools.partial(
      pl.kernel,
      out_type=jax.ShapeDtypeStruct((T, 8, 128, 8, 128), jnp.float32),
      mesh=mesh,
      scratch_types=(
          [pltpu.VMEM((2, 128), jnp.int32) for _ in range(2)]
          + [pltpu.VMEM((256, D_MODEL), jnp.float32) for _ in range(2)]
          + [pltpu.VMEM((8, 2, 8, 128), jnp.float32) for _ in range(2)]
          + [pltpu.SemaphoreType.DMA] * 6
      ),
      compiler_params=pltpu.CompilerParams(
          use_tc_tiling_on_sc=False, needs_layout_passes=False),
  )
  def lookup(table_hbm, xp_hbm, out_hbm, *bufs):
    idx_v = bufs[0:2]
    rows_v = bufs[2:4]
    tr_v = bufs[4:6]
    sem_i = bufs[6:8]
    sem_g = bufs[8:10]
    sem_o = bufs[10:12]

    wid = lax.axis_index("s") * NUM_CORES + lax.axis_index("c")
    p0 = wid * PAIRS_PER_W

    def coords(p):
      gp = p0 + p
      t = gp // 64
      bhi = 2 * (gp % 64)
      return t // 8, t % 8, t, bhi

    def fire_idx(p, q):
      thi, tlo, _, bhi = coords(p)
      pltpu.async_copy(xp_hbm.at[thi, bhi, tlo], idx_v[q].at[0], sem_i[q])
      pltpu.async_copy(xp_hbm.at[thi, bhi + 1, tlo], idx_v[q].at[1], sem_i[q])

    def wait_idx(p, q):
      thi, tlo, _, bhi = coords(p)
      pltpu.make_async_copy(
          xp_hbm.at[thi, bhi, tlo], idx_v[q].at[0], sem_i[q]).wait()
      pltpu.make_async_copy(
          xp_hbm.at[thi, bhi + 1, tlo], idx_v[q].at[1], sem_i[q]).wait()

    def fire_gathers(q, r):
      for u in range(2):
        pltpu.async_copy(
            table_hbm.at[idx_v[q].at[u]],
            rows_v[r].at[pl.ds(u * 128, 128)],
            sem_g[r],
        )

    def wait_gathers(q, r):
      for u in range(2):
        pltpu.make_async_copy(
            table_hbm.at[idx_v[q].at[u]],
            rows_v[r].at[pl.ds(u * 128, 128)],
            sem_g[r],
        ).wait()

    def transpose(r, ids):
      rows = rows_v[r]
      tr = tr_v[r]

      @plsc.parallel_loop(0, 8, carry=ids, unroll=2)
      def dbody(dhi, carry):
        colhi = jnp.full((16,), 0, jnp.int32) + 8 * dhi
        for dlo in range(8):
          col = colhi + dlo
          for k in range(16):
            v = plsc.load_gather(rows, [carry[k], col])
            tr[dhi, k // 8, dlo, pl.ds(16 * (k % 8), 16)] = v
        return carry

    def fire_outs(p, r):
      _, _, t, bhi = coords(p)
      for dhi in range(8):
        pltpu.async_copy(
            tr_v[r].at[dhi], out_hbm.at[t, dhi, pl.ds(bhi, 2)], sem_o[r])

    def wait_outs(p, r):
      _, _, t, bhi = coords(p)
      for dhi in range(8):
        pltpu.make_async_copy(
            tr_v[r].at[dhi], out_hbm.at[t, dhi, pl.ds(bhi, 2)], sem_o[r]
        ).wait()

    iota = lax.iota(jnp.int32, 16)
    ids = tuple(iota + 16 * k for k in range(16))

    fire_idx(0, 0)
    wait_idx(0, 0)
    fire_gathers(0, 0)
    fire_idx(1, 1)

    def body(g, carry):
      for s in range(2):
        p = 2 * g + 1 + s
        pb = (1 + s) % 2
        wait_idx(p, pb)
        fire_gathers(pb, pb)
        wait_gathers(1 - pb, 1 - pb)
        fire_idx(p + 1, 1 - pb)

        @pl.when(p >= 3)
        def _():
          wait_outs(p - 3, 1 - pb)

        transpose(1 - pb, carry)
        fire_outs(p - 1, 1 - pb)
      return carry

    lax.fori_loop(0, (PAIRS_PER_W - 2) // 2, body, ids, unroll=False)

    p = PAIRS_PER_W - 1
    wait_idx(p, 1)
    fire_gathers(1, 1)
    wait_gathers(0, 0)
    wait_outs(p - 3, 0)
    transpose(0, ids)
    fire_outs(p - 1, 0)
    wait_gathers(1, 1)
    wait_outs(p - 2, 1)
    transpose(1, ids)
    fire_outs(p, 1)
    wait_outs(p - 1, 0)
    wait_outs(p, 1)

  return lookup


@jax.jit
def kernel(x, table):
  xp = (
      x.astype(jnp.int32)
      .reshape(BHI, 128, THI, 8)
      .transpose(2, 0, 3, 1)
  )
  o5 = _build_lookup()(table, xp)
  return o5.transpose(2, 4, 0, 1, 3).reshape(B, T, D_MODEL)


# speedup vs baseline: 1.4746x; 1.4746x over previous
import functools

import jax
import jax.numpy as jnp
from jax import lax
from jax.experimental import pallas as pl
from jax.experimental.pallas import tpu as pltpu
from jax.experimental.pallas import tpu_sc as plsc

D_MODEL = 64
NUM_CORES = 2
NUM_SUBCORES = 16
NUM_WORKERS = NUM_CORES * NUM_SUBCORES

B = 16384
T = 200
BHI = B // 128
THI = T // 8
PAIRS = T * BHI // 2
PAIRS_PER_W = PAIRS // NUM_WORKERS


def _build_lookup():
  mesh = plsc.VectorSubcoreMesh(core_axis_name="c", subcore_axis_name="s")

  @functools.partial(
      pl.kernel,
      out_type=jax.ShapeDtypeStruct((T, 8, 128, 8, 128), jnp.float32),
      mesh=mesh,
      scratch_types=(
          [pltpu.VMEM((2, 128), jnp.int32) for _ in range(2)]
          + [pltpu.VMEM((256, 65), jnp.float32) for _ in range(2)]
          + [pltpu.VMEM((8, 2, 8, 128), jnp.float32) for _ in range(2)]
          + [pltpu.SemaphoreType.DMA] * 6
      ),
      compiler_params=pltpu.CompilerParams(
          use_tc_tiling_on_sc=False, needs_layout_passes=False),
  )
  def lookup(table_hbm, xp_hbm, out_hbm, *bufs):
    idx_v = bufs[0:2]
    rows_v = bufs[2:4]
    tr_v = bufs[4:6]
    sem_i = bufs[6:8]
    sem_g = bufs[8:10]
    sem_o = bufs[10:12]

    wid = lax.axis_index("s") * NUM_CORES + lax.axis_index("c")
    p0 = wid * PAIRS_PER_W

    def coords(p):
      gp = p0 + p
      t = gp // 64
      bhi = 2 * (gp % 64)
      return t // 8, t % 8, t, bhi

    def fire_idx(p, q):
      thi, tlo, _, bhi = coords(p)
      pltpu.async_copy(xp_hbm.at[thi, bhi, tlo], idx_v[q].at[0], sem_i[q])
      pltpu.async_copy(xp_hbm.at[thi, bhi + 1, tlo], idx_v[q].at[1], sem_i[q])

    def wait_idx(p, q):
      thi, tlo, _, bhi = coords(p)
      pltpu.make_async_copy(
          xp_hbm.at[thi, bhi, tlo], idx_v[q].at[0], sem_i[q]).wait()
      pltpu.make_async_copy(
          xp_hbm.at[thi, bhi + 1, tlo], idx_v[q].at[1], sem_i[q]).wait()

    def fire_gathers(q, r):
      for u in range(2):
        pltpu.async_copy(
            table_hbm.at[idx_v[q].at[u]],
            rows_v[r].at[pl.ds(u * 128, 128)],
            sem_g[r],
        )

    def wait_gathers(q, r):
      for u in range(2):
        pltpu.make_async_copy(
            table_hbm.at[idx_v[q].at[u]],
            rows_v[r].at[pl.ds(u * 128, 128)],
            sem_g[r],
        ).wait()

    def transpose(r, ids):
      rows = rows_v[r]
      tr = tr_v[r]

      @plsc.parallel_loop(0, 8, carry=ids, unroll=2)
      def dbody(dhi, carry):
        colhi = jnp.full((16,), 0, jnp.int32) + 8 * dhi
        for dlo in range(8):
          col = colhi + dlo
          for k in range(16):
            v = plsc.load_gather(rows, [carry[k], col])
            tr[dhi, k // 8, dlo, pl.ds(16 * (k % 8), 16)] = v
        return carry

    def fire_outs(p, r):
      _, _, t, bhi = coords(p)
      for dhi in range(8):
        pltpu.async_copy(
            tr_v[r].at[dhi], out_hbm.at[t, dhi, pl.ds(bhi, 2)], sem_o[r])

    def wait_outs(p, r):
      _, _, t, bhi = coords(p)
      for dhi in range(8):
        pltpu.make_async_copy(
            tr_v[r].at[dhi], out_hbm.at[t, dhi, pl.ds(bhi, 2)], sem_o[r]
        ).wait()

    iota = lax.iota(jnp.int32, 16)
    ids = tuple(iota + 16 * k for k in range(16))

    fire_idx(0, 0)
    wait_idx(0, 0)
    fire_gathers(0, 0)
    fire_idx(1, 1)

    def body(g, carry):
      for s in range(2):
        p = 2 * g + 1 + s
        pb = (1 + s) % 2
        wait_idx(p, pb)
        fire_gathers(pb, pb)
        wait_gathers(1 - pb, 1 - pb)
        fire_idx(p + 1, 1 - pb)

        @pl.when(p >= 3)
        def _():
          wait_outs(p - 3, 1 - pb)

        transpose(1 - pb, carry)
        fire_outs(p - 1, 1 - pb)
      return carry

    lax.fori_loop(0, (PAIRS_PER_W - 2) // 2, body, ids, unroll=False)

    p = PAIRS_PER_W - 1
    wait_idx(p, 1)
    fire_gathers(1, 1)
    wait_gathers(0, 0)
    wait_outs(p - 3, 0)
    transpose(0, ids)
    fire_outs(p - 1, 0)
    wait_gathers(1, 1)
    wait_outs(p - 2, 1)
    transpose(1, ids)
    fire_outs(p, 1)
    wait_outs(p - 1, 0)
    wait_outs(p, 1)

  return lookup


@jax.jit
def kernel(x, table):
  xp = (
      x.astype(jnp.int32)
      .reshape(BHI, 128, THI, 8)
      .transpose(2, 0, 3, 1)
  )
  tablep = jnp.pad(table, ((0, 0), (0, 1)))
  o5 = _build_lookup()(tablep, xp)
  return o5.transpose(2, 4, 0, 1, 3).reshape(B, T, D_MODEL)


# speedup vs baseline: 1.4770x; 1.0016x over previous
import functools

import jax
import jax.numpy as jnp
from jax import lax
from jax.experimental import pallas as pl
from jax.experimental.pallas import tpu as pltpu
from jax.experimental.pallas import tpu_sc as plsc

D_MODEL = 64
NUM_CORES = 2
NUM_SUBCORES = 16
NUM_WORKERS = NUM_CORES * NUM_SUBCORES

B = 16384
T = 200
BHI = B // 128
THI = T // 8
PAIRS = T * BHI // 2
PAIRS_PER_W = PAIRS // NUM_WORKERS


def _build_lookup():
  mesh = plsc.VectorSubcoreMesh(core_axis_name="c", subcore_axis_name="s")

  @functools.partial(
      pl.kernel,
      out_type=jax.ShapeDtypeStruct((T, 8, 128, 8, 128), jnp.float32),
      mesh=mesh,
      scratch_types=(
          [pltpu.VMEM((2, 128), jnp.int32) for _ in range(2)]
          + [pltpu.VMEM((256, 72), jnp.float32) for _ in range(2)]
          + [pltpu.VMEM((8, 2, 8, 128), jnp.float32) for _ in range(2)]
          + [pltpu.SemaphoreType.DMA] * 6
      ),
      compiler_params=pltpu.CompilerParams(
          use_tc_tiling_on_sc=False, needs_layout_passes=False),
  )
  def lookup(table_hbm, xp_hbm, out_hbm, *bufs):
    idx_v = bufs[0:2]
    rows_v = bufs[2:4]
    tr_v = bufs[4:6]
    sem_i = bufs[6:8]
    sem_g = bufs[8:10]
    sem_o = bufs[10:12]

    wid = lax.axis_index("s") * NUM_CORES + lax.axis_index("c")
    p0 = wid * PAIRS_PER_W

    def coords(p):
      gp = p0 + p
      t = gp // 64
      bhi = 2 * (gp % 64)
      return t // 8, t % 8, t, bhi

    def fire_idx(p, q):
      thi, tlo, _, bhi = coords(p)
      pltpu.async_copy(xp_hbm.at[thi, bhi, tlo], idx_v[q].at[0], sem_i[q])
      pltpu.async_copy(xp_hbm.at[thi, bhi + 1, tlo], idx_v[q].at[1], sem_i[q])

    def wait_idx(p, q):
      thi, tlo, _, bhi = coords(p)
      pltpu.make_async_copy(
          xp_hbm.at[thi, bhi, tlo], idx_v[q].at[0], sem_i[q]).wait()
      pltpu.make_async_copy(
          xp_hbm.at[thi, bhi + 1, tlo], idx_v[q].at[1], sem_i[q]).wait()

    def fire_gathers(q, r):
      for u in range(2):
        pltpu.async_copy(
            table_hbm.at[idx_v[q].at[u]],
            rows_v[r].at[pl.ds(u * 128, 128)],
            sem_g[r],
        )

    def wait_gathers(q, r):
      for u in range(2):
        pltpu.make_async_copy(
            table_hbm.at[idx_v[q].at[u]],
            rows_v[r].at[pl.ds(u * 128, 128)],
            sem_g[r],
        ).wait()

    def transpose(r, ids):
      rows = rows_v[r]
      tr = tr_v[r]

      @plsc.parallel_loop(0, 8, carry=ids, unroll=2)
      def dbody(dhi, carry):
        colhi = jnp.full((16,), 0, jnp.int32) + 8 * dhi
        for dlo in range(8):
          col = colhi + dlo
          for k in range(16):
            v = plsc.load_gather(rows, [carry[k], col])
            tr[dhi, k // 8, dlo, pl.ds(16 * (k % 8), 16)] = v
        return carry

    def fire_outs(p, r):
      _, _, t, bhi = coords(p)
      for dhi in range(8):
        pltpu.async_copy(
            tr_v[r].at[dhi], out_hbm.at[t, dhi, pl.ds(bhi, 2)], sem_o[r])

    def wait_outs(p, r):
      _, _, t, bhi = coords(p)
      for dhi in range(8):
        pltpu.make_async_copy(
            tr_v[r].at[dhi], out_hbm.at[t, dhi, pl.ds(bhi, 2)], sem_o[r]
        ).wait()

    iota = lax.iota(jnp.int32, 16)
    ids = tuple(iota + 16 * k for k in range(16))

    fire_idx(0, 0)
    wait_idx(0, 0)
    fire_gathers(0, 0)
    fire_idx(1, 1)

    def body(g, carry):
      for s in range(2):
        p = 2 * g + 1 + s
        pb = (1 + s) % 2
        wait_idx(p, pb)
        fire_gathers(pb, pb)
        wait_gathers(1 - pb, 1 - pb)
        fire_idx(p + 1, 1 - pb)

        @pl.when(p >= 3)
        def _():
          wait_outs(p - 3, 1 - pb)

        transpose(1 - pb, carry)
        fire_outs(p - 1, 1 - pb)
      return carry

    lax.fori_loop(0, (PAIRS_PER_W - 2) // 2, body, ids, unroll=False)

    p = PAIRS_PER_W - 1
    wait_idx(p, 1)
    fire_gathers(1, 1)
    wait_gathers(0, 0)
    wait_outs(p - 3, 0)
    transpose(0, ids)
    fire_outs(p - 1, 0)
    wait_gathers(1, 1)
    wait_outs(p - 2, 1)
    transpose(1, ids)
    fire_outs(p, 1)
    wait_outs(p - 1, 0)
    wait_outs(p, 1)

  return lookup


@jax.jit
def kernel(x, table):
  xp = (
      x.astype(jnp.int32)
      .reshape(BHI, 128, THI, 8)
      .transpose(2, 0, 3, 1)
  )
  tablep = jnp.pad(table, ((0, 0), (0, 8)))
  o5 = _build_lookup()(tablep, xp)
  return o5.transpose(2, 4, 0, 1, 3).reshape(B, T, D_MODEL)
